# Initial kernel scaffold; baseline (speedup 1.0000x reference)
import jax
import jax.numpy as jnp
from jax.experimental import pallas as pl


def kernel(x, table):
    raise NotImplementedError("write your pallas kernel here")


# speedup vs baseline: 48.3233x; 48.3233x over previous
import functools

import jax
import jax.numpy as jnp
from jax import lax
from jax.experimental import pallas as pl
from jax.experimental.pallas import tpu as pltpu
from jax.experimental.pallas import tpu_sc as plsc

_RNG = 7
_SIDE = _RNG + 1
_PAD_IDX = _SIDE * _SIDE


@functools.lru_cache(maxsize=None)
def _build_sc_fn(H, nx, ny):
    N = nx * ny
    NC, NS = 2, 16
    NW = NC * NS
    PAIRS = H * ny
    PPW = PAIRS // NW
    FLEN = (2 * _RNG + 1) * ny
    ZPRE = ny * (nx - 1 - _RNG)
    ZLEN = ZPRE + _RNG * ny + N

    mesh = plsc.VectorSubcoreMesh(core_axis_name="c", subcore_axis_name="s")

    @functools.partial(
        pl.kernel,
        mesh=mesh,
        out_type=jax.ShapeDtypeStruct((H, N, N), jnp.float32),
        scratch_types=[
            pltpu.VMEM(((_PAD_IDX + 1) * H,), jnp.float32),
            pltpu.VMEM((PPW * ZLEN,), jnp.float32),
            pltpu.SemaphoreType.DMA,
        ],
        compiler_params=pltpu.CompilerParams(
            needs_layout_passes=False, use_tc_tiling_on_sc=False
        ),
    )
    def sc(table_hbm, out_hbm, tab_v, z_v, sem):
        cid = lax.axis_index("c")
        sid = lax.axis_index("s")
        wid = sid * NC + cid
        lane = lax.iota(jnp.int32, 16)

        pltpu.sync_copy(table_hbm, tab_v)

        def zero_body(j, carry):
            z_v[pl.ds(j * 16, 16)] = jnp.zeros((16,), jnp.float32)
            return carry

        lax.fori_loop(0, PPW * ZLEN // 16, zero_body, 0)

        for t in range(PPW):
            pair = wid * PPW + t
            h = pair // ny
            yi = pair % ny
            hvec = jnp.zeros((16,), jnp.int32) + h

            def f_body(v, carry):
                tt = lane + v * 16
                a = jnp.abs((tt >> 5) - _RNG)
                b = jnp.abs(yi - (tt & (ny - 1)))
                tok = jnp.where(b <= _RNG, a * _SIDE + b, _PAD_IDX)
                vals = plsc.load_gather(tab_v, [tok * H + hvec])
                z_v[pl.ds(t * ZLEN + ZPRE + v * 16, 16)] = vals
                return carry

            lax.fori_loop(0, FLEN // 16, f_body, 0)

        def out_body(xi, carry):
            cps = []
            for t in range(PPW):
                pair = wid * PPW + t
                h = pair // ny
                yi = pair % ny
                row = xi * ny + yi
                off = pl.multiple_of(
                    t * ZLEN + ZPRE + _RNG * ny - ny * xi, ny
                )
                cps.append(
                    pltpu.async_copy(
                        z_v.at[pl.ds(off, N)], out_hbm.at[h, row], sem
                    )
                )
            for cp in cps:
                cp.wait()
            return carry

        lax.fori_loop(0, nx, out_body, 0)

    return sc


def kernel(x, table):
    nx, ny = x.shape[-2], x.shape[-1]
    H = table.shape[1]
    fn = _build_sc_fn(H, nx, ny)
    return fn(table.reshape(-1))


# speedup vs baseline: 49.6513x; 1.0275x over previous
import functools

import jax
import jax.numpy as jnp
from jax import lax
from jax.experimental import pallas as pl
from jax.experimental.pallas import tpu as pltpu
from jax.experimental.pallas import tpu_sc as plsc

_RNG = 7
_SIDE = _RNG + 1
_PAD_IDX = _SIDE * _SIDE


@functools.lru_cache(maxsize=None)
def _build_sc_fn(H, nx, ny):
    N = nx * ny
    NC, NS = 2, 16
    NW = NC * NS
    PAIRS = H * ny
    PPW = PAIRS // NW
    FLEN = (2 * _RNG + 1) * ny
    ZPRE = ny * (nx - 1 - _RNG)
    ZLEN = ZPRE + _RNG * ny + N

    mesh = plsc.VectorSubcoreMesh(core_axis_name="c", subcore_axis_name="s")

    @functools.partial(
        pl.kernel,
        mesh=mesh,
        out_type=jax.ShapeDtypeStruct((H, N, N), jnp.float32),
        scratch_types=[
            pltpu.VMEM(((_PAD_IDX + 1) * H,), jnp.float32),
            pltpu.VMEM((PPW * ZLEN,), jnp.float32),
            pltpu.SemaphoreType.DMA,
        ],
        compiler_params=pltpu.CompilerParams(
            needs_layout_passes=False, use_tc_tiling_on_sc=False
        ),
    )
    def sc(table_hbm, out_hbm, tab_v, z_v, sem):
        cid = lax.axis_index("c")
        sid = lax.axis_index("s")
        wid = sid * NC + cid
        lane = lax.iota(jnp.int32, 16)

        pltpu.sync_copy(table_hbm, tab_v)

        def zero_body(j, carry):
            z_v[pl.ds(j * 16, 16)] = jnp.zeros((16,), jnp.float32)
            return carry

        lax.fori_loop(0, PPW * ZLEN // 16, zero_body, 0)

        for t in range(PPW):
            pair = wid * PPW + t
            h = pair // ny
            yi = pair % ny
            hvec = jnp.zeros((16,), jnp.int32) + h

            def f_body(v, carry):
                tt = lane + v * 16
                a = jnp.abs((tt >> 5) - _RNG)
                b = jnp.abs(yi - (tt & (ny - 1)))
                tok = jnp.where(b <= _RNG, a * _SIDE + b, _PAD_IDX)
                vals = plsc.load_gather(tab_v, [tok * H + hvec])
                z_v[pl.ds(t * ZLEN + ZPRE + v * 16, 16)] = vals
                return carry

            lax.fori_loop(0, FLEN // 16, f_body, 0)

        def out_body(xi, carry):
            for t in range(PPW):
                pair = wid * PPW + t
                h = pair // ny
                yi = pair % ny
                row = xi * ny + yi
                off = pl.multiple_of(
                    t * ZLEN + ZPRE + _RNG * ny - ny * xi, ny
                )
                pltpu.async_copy(
                    z_v.at[pl.ds(off, N)], out_hbm.at[h, row], sem
                )
            return carry

        lax.fori_loop(0, nx, out_body, 0)

        def drain_body(j, carry):
            pltpu.make_async_copy(
                z_v.at[pl.ds(0, N)], out_hbm.at[0, 0], sem
            ).wait()
            return carry

        lax.fori_loop(0, nx * PPW, drain_body, 0)

    return sc


def kernel(x, table):
    nx, ny = x.shape[-2], x.shape[-1]
    H = table.shape[1]
    fn = _build_sc_fn(H, nx, ny)
    return fn(table.reshape(-1))
